# baseline (speedup 1.0000x reference)
import functools

import jax
import jax.numpy as jnp
import numpy as np
from jax import lax
from jax.experimental import pallas as pl
from jax.experimental.pallas import tpu as pltpu
from jax.experimental.pallas import tpu_sc as plsc

N = 10000
E = 160000
MAXNB = 8
DEPTH = 3
H = 256
NW = 32
HP = H // 2
_HI = np.uint32(0xFFFF0000)
_RND = np.uint32(0x8000)

_MESH = plsc.VectorSubcoreMesh(core_axis_name="c", subcore_axis_name="s",
                               num_cores=2, num_subcores=16)


def _wid():
    return lax.axis_index("s") * 2 + lax.axis_index("c")


def _ceil_div(a, b):
    return (a + b - 1) // b


def sc_gathersum(table, idx_flat):
    CH = 16
    IB = CH * 8
    NBLK = N // CH
    G = _ceil_div(NBLK, NW)
    GP = _ceil_div(G, 2)

    def body(table_hbm, idx_hbm, out_hbm, idxa, rows0, rows1, acc0, acc1,
             semi, semg0, semg1, semw0, semw1):
        w = _wid()
        rows = (rows0, rows1)
        acc = (acc0, acc1)
        semg = (semg0, semg1)
        semw = (semw0, semw1)

        def idx_sl(g):
            return idxa.at[pl.ds(g * IB, IB)]

        def gather_cp(g, p):
            return pltpu.make_async_copy(table_hbm.at[idx_sl(g)],
                                         rows[p], semg[p])

        def wb_cp(bg, p):
            return pltpu.make_async_copy(acc[p],
                                         out_hbm.at[pl.ds(bg * CH, CH)],
                                         semw[p])

        def fire_idx(g, _):
            bg = w + g * NW

            @pl.when(bg < NBLK)
            def _():
                pltpu.async_copy(idx_hbm.at[pl.ds(bg * IB, IB)],
                                 idx_sl(g), semi)
            return 0

        lax.fori_loop(0, G, fire_idx, 0)

        def drain_idx(g, _):
            bg = w + g * NW

            @pl.when(bg < NBLK)
            def _():
                pltpu.make_async_copy(idx_hbm.at[pl.ds(bg * IB, IB)],
                                      idx_sl(g), semi).wait()
            return 0

        lax.fori_loop(0, G, drain_idx, 0)

        gather_cp(0, 0).start()

        def compute(g, p):
            def atom(a, _2):
                base = a * 8
                for t in range(HP // 16):
                    sl = pl.ds(t * 16, 16)
                    s0 = jnp.zeros((16,), jnp.float32)
                    s1 = jnp.zeros((16,), jnp.float32)
                    for j in range(8):
                        wv = rows[p][base + j, sl]
                        s0 = s0 + lax.bitcast_convert_type(wv << 16, jnp.float32)
                        s1 = s1 + lax.bitcast_convert_type(wv & _HI, jnp.float32)
                    u0 = (lax.bitcast_convert_type(s0, jnp.uint32) + _RND) >> 16
                    u1 = (lax.bitcast_convert_type(s1, jnp.uint32) + _RND) & _HI
                    acc[p][a, sl] = u0 | u1
                return 0

            lax.fori_loop(0, CH, atom, 0)

        def pair(q, _):
            for p in (0, 1):
                g = q * 2 + p
                bg = w + g * NW
                bn = bg + NW

                @pl.when(bn < NBLK)
                def _():
                    gather_cp(g + 1, p ^ 1).start()

                @pl.when(bg < NBLK)
                def _():
                    gather_cp(g, p).wait()

                    @pl.when(g >= 2)
                    def _():
                        wb_cp(bg - 2 * NW, p).wait()

                    compute(g, p)
                    wb_cp(bg, p).start()
            return 0

        lax.fori_loop(0, GP, pair, 0)

        nv = (NBLK - 1 - w) // NW + 1
        for p in (0, 1):
            gl = nv - 1 - (((nv - 1) ^ p) & 1)
            bg = w + gl * NW

            @pl.when(gl >= 0)
            def _():
                wb_cp(bg, p).wait()

    f = pl.kernel(
        body,
        out_type=jax.ShapeDtypeStruct((N, HP), jnp.uint32),
        mesh=_MESH,
        scratch_types=[
            pltpu.VMEM((G * IB,), jnp.int32),
            pltpu.VMEM((IB, HP), jnp.uint32),
            pltpu.VMEM((IB, HP), jnp.uint32),
            pltpu.VMEM((CH, HP), jnp.uint32),
            pltpu.VMEM((CH, HP), jnp.uint32),
            pltpu.SemaphoreType.DMA,
            pltpu.SemaphoreType.DMA,
            pltpu.SemaphoreType.DMA,
            pltpu.SemaphoreType.DMA,
            pltpu.SemaphoreType.DMA,
        ],
    )
    return f(table, idx_flat)


def sc_msg(nei, h, b2a, b2revb):
    CB = 128
    NBLK = E // CB
    G = _ceil_div(NBLK, NW)
    GT = _ceil_div(G, 3)

    def body(nei_hbm, h_hbm, b2a_hbm, brev_hbm, outa_hbm, outb_hbm,
             ia, ib, ra0, ra1, ra2, rb0, rb1, rb2,
             semi, semga0, semga1, semga2, semgb0, semgb1, semgb2,
             semwa0, semwa1, semwa2, semwb0, semwb1, semwb2):
        w = _wid()
        ra = (ra0, ra1, ra2)
        rb = (rb0, rb1, rb2)
        semga = (semga0, semga1, semga2)
        semgb = (semgb0, semgb1, semgb2)
        semwa = (semwa0, semwa1, semwa2)
        semwb = (semwb0, semwb1, semwb2)

        def ga_cp(g, p):
            return pltpu.make_async_copy(
                nei_hbm.at[ia.at[pl.ds(g * CB, CB)]], ra[p], semga[p])

        def gb_cp(g, p):
            return pltpu.make_async_copy(
                h_hbm.at[ib.at[pl.ds(g * CB, CB)]], rb[p], semgb[p])

        def wa_cp(bg, p):
            return pltpu.make_async_copy(ra[p],
                                         outa_hbm.at[pl.ds(bg * CB, CB)],
                                         semwa[p])

        def wb_cp(bg, p):
            return pltpu.make_async_copy(rb[p],
                                         outb_hbm.at[pl.ds(bg * CB, CB)],
                                         semwb[p])

        def fire_idx(g, _):
            bg = w + g * NW

            @pl.when(bg < NBLK)
            def _():
                pltpu.async_copy(b2a_hbm.at[pl.ds(bg * CB, CB)],
                                 ia.at[pl.ds(g * CB, CB)], semi)
                pltpu.async_copy(brev_hbm.at[pl.ds(bg * CB, CB)],
                                 ib.at[pl.ds(g * CB, CB)], semi)
            return 0

        lax.fori_loop(0, G, fire_idx, 0)

        def drain_idx(g, _):
            bg = w + g * NW

            @pl.when(bg < NBLK)
            def _():
                pltpu.make_async_copy(b2a_hbm.at[pl.ds(bg * CB, CB)],
                                      ia.at[pl.ds(g * CB, CB)], semi).wait()
                pltpu.make_async_copy(brev_hbm.at[pl.ds(bg * CB, CB)],
                                      ib.at[pl.ds(g * CB, CB)], semi).wait()
            return 0

        lax.fori_loop(0, G, drain_idx, 0)

        ga_cp(0, 0).start()
        gb_cp(0, 0).start()

        def tri(q, _):
            for p in (0, 1, 2):
                g = q * 3 + p
                bg = w + g * NW
                bn = bg + NW
                pn = (p + 1) % 3

                @pl.when(bn < NBLK)
                def _():
                    @pl.when(g >= 2)
                    def _():
                        wa_cp(bg - 2 * NW, pn).wait()
                        wb_cp(bg - 2 * NW, pn).wait()

                    ga_cp(g + 1, pn).start()
                    gb_cp(g + 1, pn).start()

                @pl.when(bg < NBLK)
                def _():
                    ga_cp(g, p).wait()
                    gb_cp(g, p).wait()
                    wa_cp(bg, p).start()
                    wb_cp(bg, p).start()
            return 0

        lax.fori_loop(0, GT, tri, 0)

        nv = (NBLK - 1 - w) // NW + 1
        for p in (0, 1, 2):
            r = (nv - 1) % 3
            gl = nv - 1 - ((r - p) % 3)

            @pl.when(gl >= 0)
            def _():
                bg = w + gl * NW
                wa_cp(bg, p).wait()
                wb_cp(bg, p).wait()

    f = pl.kernel(
        body,
        out_type=[jax.ShapeDtypeStruct((E, HP), jnp.uint32),
                  jax.ShapeDtypeStruct((E, HP), jnp.uint32)],
        mesh=_MESH,
        scratch_types=(
            [pltpu.VMEM((G * CB,), jnp.int32)] * 2
            + [pltpu.VMEM((CB, HP), jnp.uint32)] * 6
            + [pltpu.SemaphoreType.DMA] * 13
        ),
    )
    return f(nei, h, b2a, b2revb)


def sc_wsum(weighted, a2b_flat, b2a):
    CH = 16
    IB = CH * 8
    NBLK = N // CH
    G = _ceil_div(NBLK, NW)
    GP = _ceil_div(G, 2)

    def body(w_hbm, a2b_hbm, b2a_hbm, out_hbm,
             idxa, idx2a, rows0, rows1, self0, self1, acc0, acc1,
             semi, semi2, semg0, semg1, sems0, sems1, semw0, semw1):
        w = _wid()
        rows = (rows0, rows1)
        selfv = (self0, self1)
        acc = (acc0, acc1)
        semg = (semg0, semg1)
        sems = (sems0, sems1)
        semw = (semw0, semw1)

        def gather_cp(g, p):
            return pltpu.make_async_copy(
                w_hbm.at[idx2a.at[pl.ds(g * IB, IB)]], rows[p], semg[p])

        def self_cp(bg, p):
            return pltpu.make_async_copy(
                w_hbm.at[pl.ds(bg * CH, CH)], selfv[p], sems[p])

        def wb_cp(bg, p):
            return pltpu.make_async_copy(acc[p],
                                         out_hbm.at[pl.ds(bg * CH, CH)],
                                         semw[p])

        def fire_idx(g, _):
            bg = w + g * NW

            @pl.when(bg < NBLK)
            def _():
                pltpu.async_copy(a2b_hbm.at[pl.ds(bg * IB, IB)],
                                 idxa.at[pl.ds(g * IB, IB)], semi)
            return 0

        lax.fori_loop(0, G, fire_idx, 0)

        def drain_fire2(g, _):
            bg = w + g * NW

            @pl.when(bg < NBLK)
            def _():
                pltpu.make_async_copy(a2b_hbm.at[pl.ds(bg * IB, IB)],
                                      idxa.at[pl.ds(g * IB, IB)], semi).wait()
                pltpu.async_copy(b2a_hbm.at[idxa.at[pl.ds(g * IB, IB)]],
                                 idx2a.at[pl.ds(g * IB, IB)], semi2)
            return 0

        lax.fori_loop(0, G, drain_fire2, 0)

        def drain2(g, _):
            bg = w + g * NW

            @pl.when(bg < NBLK)
            def _():
                pltpu.make_async_copy(b2a_hbm.at[idxa.at[pl.ds(g * IB, IB)]],
                                      idx2a.at[pl.ds(g * IB, IB)],
                                      semi2).wait()
            return 0

        lax.fori_loop(0, G, drain2, 0)

        gather_cp(0, 0).start()
        self_cp(w, 0).start()

        def compute(g, p, bg):
            def atom(a, _2):
                base = a * 8
                for t in range(H // 16):
                    sl = pl.ds(t * 16, 16)
                    v = selfv[p][a, sl]
                    for j in range(8):
                        v = v + rows[p][base + j, sl]
                    acc[p][a, sl] = v
                return 0

            lax.fori_loop(0, CH, atom, 0)

            @pl.when(bg == 0)
            def _():
                for t in range(H // 16):
                    acc[p][0, pl.ds(t * 16, 16)] = jnp.zeros((16,),
                                                             jnp.float32)

        def pair(q, _):
            for p in (0, 1):
                g = q * 2 + p
                bg = w + g * NW
                bn = bg + NW

                @pl.when(bn < NBLK)
                def _():
                    gather_cp(g + 1, p ^ 1).start()
                    self_cp(bn, p ^ 1).start()

                @pl.when(bg < NBLK)
                def _():
                    gather_cp(g, p).wait()
                    self_cp(bg, p).wait()

                    @pl.when(g >= 2)
                    def _():
                        wb_cp(bg - 2 * NW, p).wait()

                    compute(g, p, bg)
                    wb_cp(bg, p).start()
            return 0

        lax.fori_loop(0, GP, pair, 0)

        nv = (NBLK - 1 - w) // NW + 1
        for p in (0, 1):
            gl = nv - 1 - (((nv - 1) ^ p) & 1)
            bg = w + gl * NW

            @pl.when(gl >= 0)
            def _():
                wb_cp(bg, p).wait()

    f = pl.kernel(
        body,
        out_type=jax.ShapeDtypeStruct((N, H), jnp.float32),
        mesh=_MESH,
        scratch_types=[
            pltpu.VMEM((G * IB,), jnp.int32),
            pltpu.VMEM((G * IB,), jnp.int32),
            pltpu.VMEM((IB, H), jnp.float32),
            pltpu.VMEM((IB, H), jnp.float32),
            pltpu.VMEM((CH, H), jnp.float32),
            pltpu.VMEM((CH, H), jnp.float32),
            pltpu.VMEM((CH, H), jnp.float32),
            pltpu.VMEM((CH, H), jnp.float32),
            pltpu.SemaphoreType.DMA,
            pltpu.SemaphoreType.DMA,
            pltpu.SemaphoreType.DMA,
            pltpu.SemaphoreType.DMA,
            pltpu.SemaphoreType.DMA,
            pltpu.SemaphoreType.DMA,
            pltpu.SemaphoreType.DMA,
            pltpu.SemaphoreType.DMA,
        ],
    )
    return f(weighted, a2b_flat, b2a)


def _full(shape):
    return pl.BlockSpec(shape, lambda i: (0, 0))


def _dotb(a, b):
    return jnp.dot(a.astype(jnp.bfloat16), b.astype(jnp.bfloat16),
                   preferred_element_type=jnp.float32)


def _pack_tc(y):
    tl = lax.bitcast_convert_type(y[:, :HP], jnp.uint32) + _RND
    th = lax.bitcast_convert_type(y[:, HP:], jnp.uint32) + _RND
    return (tl >> 16) | (th & _HI)


def _unpack_tc(w):
    lo = lax.bitcast_convert_type(w << 16, jnp.float32)
    hi = lax.bitcast_convert_type(w & _HI, jnp.float32)
    return jnp.concatenate([lo, hi], axis=1)


def tc_mm_relu(x, w, m_blk):
    M, K = x.shape
    Kw, Nw = w.shape

    def body(x_ref, w_ref, o_ref):
        o_ref[:] = _pack_tc(jax.nn.relu(_dotb(x_ref[:], w_ref[:])))

    return pl.pallas_call(
        body,
        grid=(M // m_blk,),
        in_specs=[pl.BlockSpec((m_blk, K), lambda i: (i, 0)), _full((Kw, Nw))],
        out_specs=pl.BlockSpec((m_blk, Nw // 2), lambda i: (i, 0)),
        out_shape=jax.ShapeDtypeStruct((M, Nw // 2), jnp.uint32),
    )(x, w)


def tc_mm_residual_relu(ga, gb, h0, w, m_blk):
    M = ga.shape[0]

    def body(a_ref, b_ref, h0_ref, w_ref, o_ref):
        msg = _unpack_tc(a_ref[:]) - _unpack_tc(b_ref[:])
        o_ref[:] = _pack_tc(jax.nn.relu(
            _unpack_tc(h0_ref[:]) + _dotb(msg, w_ref[:])))

    return pl.pallas_call(
        body,
        grid=(M // m_blk,),
        in_specs=[pl.BlockSpec((m_blk, HP), lambda i: (i, 0)),
                  pl.BlockSpec((m_blk, HP), lambda i: (i, 0)),
                  pl.BlockSpec((m_blk, HP), lambda i: (i, 0)),
                  _full((H, H))],
        out_specs=pl.BlockSpec((m_blk, HP), lambda i: (i, 0)),
        out_shape=jax.ShapeDtypeStruct((M, HP), jnp.uint32),
    )(ga, gb, h0, w)


def tc_atoms(f_atoms, nei, prev, eids, mids, table, W_o, W_vv, W_vc,
             W_conf, b_conf, W_ve_out):
    MB = 1000
    AF = f_atoms.shape[1]
    V = table.shape[0]

    def body(fa, ne, pv, ei, mi, tb, wo, wvv, wvc, wc, bc, wveo,
             atf_o, wgt_o, veo_o):
        pid = pl.program_id(0)
        wo_a = wo[:]
        ne_f = _unpack_tc(ne[:])
        af = jax.nn.relu(_dotb(fa[:], wo_a[:AF]) + _dotb(ne_f, wo_a[AF:]))
        atf = jax.nn.relu(_dotb(pv[:], wvv[:]) + _dotb(af, wvc[:]))

        glob = lax.broadcasted_iota(jnp.int32, (MB, H), 0) + pid * MB
        iota_v = lax.broadcasted_iota(jnp.int32, (MB, V), 1)
        oh_e = (ei[:] == iota_v).astype(jnp.float32)
        ee = _dotb(oh_e, tb[:])
        ee = jnp.where(glob == 0, 0.0, ee)
        oh_m = (mi[:] == iota_v).astype(jnp.float32)
        me = _dotb(oh_m, tb[:])
        me = jnp.where(glob == 0, 0.0, me)

        wc_a = wc[:]
        imp = _dotb(ee, wc_a[:H]) + _dotb(atf, wc_a[H:]) + bc[0, 0]
        atf_o[:] = atf
        wgt_o[:] = imp * ee
        veo_o[:] = _dotb(jax.nn.relu(me), wveo[:])

    return pl.pallas_call(
        body,
        grid=(N // MB,),
        in_specs=[
            pl.BlockSpec((MB, AF), lambda i: (i, 0)),
            pl.BlockSpec((MB, HP), lambda i: (i, 0)),
            pl.BlockSpec((MB, H), lambda i: (i, 0)),
            pl.BlockSpec((MB, 1), lambda i: (i, 0)),
            pl.BlockSpec((MB, 1), lambda i: (i, 0)),
            _full((V, H)),
            _full((AF + H, H)),
            _full((H, H)),
            _full((H, H)),
            _full((2 * H, 1)),
            _full((1, 1)),
            _full((H, H)),
        ],
        out_specs=[pl.BlockSpec((MB, H), lambda i: (i, 0))] * 3,
        out_shape=[jax.ShapeDtypeStruct((N, H), jnp.float32)] * 3,
    )(f_atoms, nei, prev, eids, mids, table, W_o, W_vv, W_vc, W_conf,
      b_conf, W_ve_out)


def tc_head(atf, aee, veo, W_ve, W_ea1, b_ea1, W_ea2, b_ea2,
            W_al1, b_al1, W_al2, b_al2):
    MB = 1000
    MLP = W_ea1.shape[1]
    AOUT = W_al2.shape[1]

    def body(at, ae, vo, wve, wea1, bea1, wea2, bea2, wal1, bal1,
             wal2, bal2, o_ref):
        ve = _dotb(jax.nn.relu(ae[:]), wve[:])
        wea1_a = wea1[:]
        h1 = jax.nn.relu(
            _dotb(at[:], wea1_a[:H]) + _dotb(ve, wea1_a[H:2 * H])
            + _dotb(vo[:], wea1_a[2 * H:]) + bea1[:])
        af2 = jax.nn.relu(_dotb(h1, wea2[:]) + bea2[:])
        s1 = jax.nn.relu(_dotb(af2, wal1[:]) + bal1[:])
        o_ref[:] = _dotb(s1, wal2[:]) + bal2[:]

    return pl.pallas_call(
        body,
        grid=(N // MB,),
        in_specs=[
            pl.BlockSpec((MB, H), lambda i: (i, 0)),
            pl.BlockSpec((MB, H), lambda i: (i, 0)),
            pl.BlockSpec((MB, H), lambda i: (i, 0)),
            _full((H, H)),
            _full((3 * H, MLP)),
            _full((1, MLP)),
            _full((MLP, H)),
            _full((1, H)),
            _full((H, MLP)),
            _full((1, MLP)),
            _full((MLP, AOUT)),
            _full((1, AOUT)),
        ],
        out_specs=pl.BlockSpec((MB, AOUT), lambda i: (i, 0)),
        out_shape=jax.ShapeDtypeStruct((N, AOUT), jnp.float32),
    )(atf, aee, veo, W_ve, W_ea1, b_ea1, W_ea2, b_ea2, W_al1, b_al1,
      W_al2, b_al2)


def kernel(f_atoms, f_bonds, a2b, b2a, b2revb, edit_data, last_edit_mask,
           prev_atom_hiddens, edit_table, W_i, W_h, W_o, W_vv, W_vc, W_conf,
           b_conf, W_ve, W_ve_out, W_ea1, b_ea1, W_ea2, b_ea2, W_al1, b_al1,
           W_al2, b_al2):
    a2b_flat = a2b.reshape(-1).astype(jnp.int32)
    b2a32 = b2a.astype(jnp.int32)
    b2revb32 = b2revb.astype(jnp.int32)

    h0 = tc_mm_relu(f_bonds.astype(jnp.bfloat16), W_i, 2000)
    h = h0
    for _ in range(DEPTH - 1):
        nei = sc_gathersum(h, a2b_flat)
        ga, gb = sc_msg(nei, h, b2a32, b2revb32)
        h = tc_mm_residual_relu(ga, gb, h0, W_h, 2000)
    nei = sc_gathersum(h, a2b_flat)

    atf, weighted, veo = tc_atoms(
        f_atoms, nei, prev_atom_hiddens,
        edit_data.reshape(-1, 1).astype(jnp.int32),
        last_edit_mask.reshape(-1, 1).astype(jnp.int32),
        edit_table, W_o, W_vv, W_vc, W_conf, b_conf.reshape(1, 1), W_ve_out)

    aee = sc_wsum(weighted, a2b_flat, b2a32)

    return tc_head(atf, aee, veo, W_ve, W_ea1, b_ea1.reshape(1, -1),
                   W_ea2, b_ea2.reshape(1, -1), W_al1, b_al1.reshape(1, -1),
                   W_al2, b_al2.reshape(1, -1))


# speedup vs baseline: 1.0508x; 1.0508x over previous
import functools

import jax
import jax.numpy as jnp
import numpy as np
from jax import lax
from jax.experimental import pallas as pl
from jax.experimental.pallas import tpu as pltpu
from jax.experimental.pallas import tpu_sc as plsc

N = 10000
E = 160000
MAXNB = 8
DEPTH = 3
H = 256
NW = 32
HP = H // 2
_HI = np.uint32(0xFFFF0000)
_RND = np.uint32(0x8000)

_MESH = plsc.VectorSubcoreMesh(core_axis_name="c", subcore_axis_name="s",
                               num_cores=2, num_subcores=16)


def _wid():
    return lax.axis_index("s") * 2 + lax.axis_index("c")


def _ceil_div(a, b):
    return (a + b - 1) // b


def sc_gathersum(table, idx_flat):
    CH = 16
    IB = CH * 8
    NBLK = N // CH
    G = _ceil_div(NBLK, NW)
    GP = _ceil_div(G, 2)

    def body(table_hbm, idx_hbm, out_hbm, idxa, rows0, rows1, acc0, acc1,
             semi, semg0, semg1, semw0, semw1):
        w = _wid()
        rows = (rows0, rows1)
        acc = (acc0, acc1)
        semg = (semg0, semg1)
        semw = (semw0, semw1)

        def idx_sl(g):
            return idxa.at[pl.ds(g * IB, IB)]

        def gather_cp(g, p):
            return pltpu.make_async_copy(table_hbm.at[idx_sl(g)],
                                         rows[p], semg[p])

        def wb_cp(bg, p):
            return pltpu.make_async_copy(acc[p],
                                         out_hbm.at[pl.ds(bg * CH, CH)],
                                         semw[p])

        def fire_idx(g, _):
            bg = w + g * NW

            @pl.when(bg < NBLK)
            def _():
                pltpu.async_copy(idx_hbm.at[pl.ds(bg * IB, IB)],
                                 idx_sl(g), semi)
            return 0

        lax.fori_loop(0, G, fire_idx, 0)

        def drain_idx(g, _):
            bg = w + g * NW

            @pl.when(bg < NBLK)
            def _():
                pltpu.make_async_copy(idx_hbm.at[pl.ds(bg * IB, IB)],
                                      idx_sl(g), semi).wait()
            return 0

        lax.fori_loop(0, G, drain_idx, 0)

        gather_cp(0, 0).start()

        def compute(g, p):
            def atom(a, _2):
                base = a * 8
                for t in range(HP // 16):
                    sl = pl.ds(t * 16, 16)
                    s0 = jnp.zeros((16,), jnp.float32)
                    s1 = jnp.zeros((16,), jnp.float32)
                    for j in range(8):
                        wv = rows[p][base + j, sl]
                        s0 = s0 + lax.bitcast_convert_type(wv << 16, jnp.float32)
                        s1 = s1 + lax.bitcast_convert_type(wv & _HI, jnp.float32)
                    u0 = (lax.bitcast_convert_type(s0, jnp.uint32) + _RND) >> 16
                    u1 = (lax.bitcast_convert_type(s1, jnp.uint32) + _RND) & _HI
                    acc[p][a, sl] = u0 | u1
                return 0

            lax.fori_loop(0, CH, atom, 0)

        def pair(q, _):
            for p in (0, 1):
                g = q * 2 + p
                bg = w + g * NW
                bn = bg + NW

                @pl.when(bn < NBLK)
                def _():
                    gather_cp(g + 1, p ^ 1).start()

                @pl.when(bg < NBLK)
                def _():
                    gather_cp(g, p).wait()

                    @pl.when(g >= 2)
                    def _():
                        wb_cp(bg - 2 * NW, p).wait()

                    compute(g, p)
                    wb_cp(bg, p).start()
            return 0

        lax.fori_loop(0, GP, pair, 0)

        nv = (NBLK - 1 - w) // NW + 1
        for p in (0, 1):
            gl = nv - 1 - (((nv - 1) ^ p) & 1)
            bg = w + gl * NW

            @pl.when(gl >= 0)
            def _():
                wb_cp(bg, p).wait()

    f = pl.kernel(
        body,
        out_type=jax.ShapeDtypeStruct((N, HP), jnp.uint32),
        mesh=_MESH,
        scratch_types=[
            pltpu.VMEM((G * IB,), jnp.int32),
            pltpu.VMEM((IB, HP), jnp.uint32),
            pltpu.VMEM((IB, HP), jnp.uint32),
            pltpu.VMEM((CH, HP), jnp.uint32),
            pltpu.VMEM((CH, HP), jnp.uint32),
            pltpu.SemaphoreType.DMA,
            pltpu.SemaphoreType.DMA,
            pltpu.SemaphoreType.DMA,
            pltpu.SemaphoreType.DMA,
            pltpu.SemaphoreType.DMA,
        ],
    )
    return f(table, idx_flat)


def sc_msg(nei, h, b2a, b2revb):
    CB = 64
    NBLK = E // CB
    G = _ceil_div(NBLK, NW)
    GP = _ceil_div(G, 2)

    def body(nei_hbm, h_hbm, b2a_hbm, brev_hbm, out_hbm,
             ia, ib, ra0, ra1, rb0, rb1, oc0, oc1,
             semi, semga0, semga1, semgb0, semgb1, semw0, semw1):
        w = _wid()
        ra = (ra0, ra1)
        rb = (rb0, rb1)
        oc = (oc0, oc1)
        semga = (semga0, semga1)
        semgb = (semgb0, semgb1)
        semw = (semw0, semw1)

        def ga_cp(g, p):
            return pltpu.make_async_copy(
                nei_hbm.at[ia.at[pl.ds(g * CB, CB)]], ra[p], semga[p])

        def gb_cp(g, p):
            return pltpu.make_async_copy(
                h_hbm.at[ib.at[pl.ds(g * CB, CB)]], rb[p], semgb[p])

        def wb_cp(bg, p):
            return pltpu.make_async_copy(oc[p],
                                         out_hbm.at[pl.ds(bg * CB, CB)],
                                         semw[p])

        def fire_idx(g, _):
            bg = w + g * NW

            @pl.when(bg < NBLK)
            def _():
                pltpu.async_copy(b2a_hbm.at[pl.ds(bg * CB, CB)],
                                 ia.at[pl.ds(g * CB, CB)], semi)
                pltpu.async_copy(brev_hbm.at[pl.ds(bg * CB, CB)],
                                 ib.at[pl.ds(g * CB, CB)], semi)
            return 0

        lax.fori_loop(0, G, fire_idx, 0)

        def drain_idx(g, _):
            bg = w + g * NW

            @pl.when(bg < NBLK)
            def _():
                pltpu.make_async_copy(b2a_hbm.at[pl.ds(bg * CB, CB)],
                                      ia.at[pl.ds(g * CB, CB)], semi).wait()
                pltpu.make_async_copy(brev_hbm.at[pl.ds(bg * CB, CB)],
                                      ib.at[pl.ds(g * CB, CB)], semi).wait()
            return 0

        lax.fori_loop(0, G, drain_idx, 0)

        ga_cp(0, 0).start()
        gb_cp(0, 0).start()

        def compute(g, p):
            def bond(a, _2):
                for t in range(HP // 16):
                    sl = pl.ds(t * 16, 16)
                    wa = ra[p][a, sl]
                    wb = rb[p][a, sl]
                    d0 = (lax.bitcast_convert_type(wa << 16, jnp.float32)
                          - lax.bitcast_convert_type(wb << 16, jnp.float32))
                    d1 = (lax.bitcast_convert_type(wa & _HI, jnp.float32)
                          - lax.bitcast_convert_type(wb & _HI, jnp.float32))
                    u0 = (lax.bitcast_convert_type(d0, jnp.uint32) + _RND) >> 16
                    u1 = (lax.bitcast_convert_type(d1, jnp.uint32) + _RND) & _HI
                    oc[p][a, sl] = u0 | u1
                return 0

            lax.fori_loop(0, CB, bond, 0)

        def pair(q, _):
            for p in (0, 1):
                g = q * 2 + p
                bg = w + g * NW
                bn = bg + NW

                @pl.when(bn < NBLK)
                def _():
                    ga_cp(g + 1, p ^ 1).start()
                    gb_cp(g + 1, p ^ 1).start()

                @pl.when(bg < NBLK)
                def _():
                    ga_cp(g, p).wait()
                    gb_cp(g, p).wait()

                    @pl.when(g >= 2)
                    def _():
                        wb_cp(bg - 2 * NW, p).wait()

                    compute(g, p)
                    wb_cp(bg, p).start()
            return 0

        lax.fori_loop(0, GP, pair, 0)

        nv = (NBLK - 1 - w) // NW + 1
        for p in (0, 1):
            gl = nv - 1 - (((nv - 1) ^ p) & 1)
            bg = w + gl * NW

            @pl.when(gl >= 0)
            def _():
                wb_cp(bg, p).wait()

    f = pl.kernel(
        body,
        out_type=jax.ShapeDtypeStruct((E, HP), jnp.uint32),
        mesh=_MESH,
        scratch_types=[
            pltpu.VMEM((G * CB,), jnp.int32),
            pltpu.VMEM((G * CB,), jnp.int32),
            pltpu.VMEM((CB, HP), jnp.uint32),
            pltpu.VMEM((CB, HP), jnp.uint32),
            pltpu.VMEM((CB, HP), jnp.uint32),
            pltpu.VMEM((CB, HP), jnp.uint32),
            pltpu.VMEM((CB, HP), jnp.uint32),
            pltpu.VMEM((CB, HP), jnp.uint32),
            pltpu.SemaphoreType.DMA,
            pltpu.SemaphoreType.DMA,
            pltpu.SemaphoreType.DMA,
            pltpu.SemaphoreType.DMA,
            pltpu.SemaphoreType.DMA,
            pltpu.SemaphoreType.DMA,
            pltpu.SemaphoreType.DMA,
        ],
    )
    return f(nei, h, b2a, b2revb)


def sc_wsum(weighted, a2b_flat, b2a):
    CH = 16
    IB = CH * 8
    NBLK = N // CH
    G = _ceil_div(NBLK, NW)
    GP = _ceil_div(G, 2)

    def body(w_hbm, a2b_hbm, b2a_hbm, out_hbm,
             idxa, idx2a, rows0, rows1, self0, self1, acc0, acc1,
             semi, semi2, semg0, semg1, sems0, sems1, semw0, semw1):
        w = _wid()
        rows = (rows0, rows1)
        selfv = (self0, self1)
        acc = (acc0, acc1)
        semg = (semg0, semg1)
        sems = (sems0, sems1)
        semw = (semw0, semw1)

        def gather_cp(g, p):
            return pltpu.make_async_copy(
                w_hbm.at[idx2a.at[pl.ds(g * IB, IB)]], rows[p], semg[p])

        def self_cp(bg, p):
            return pltpu.make_async_copy(
                w_hbm.at[pl.ds(bg * CH, CH)], selfv[p], sems[p])

        def wb_cp(bg, p):
            return pltpu.make_async_copy(acc[p],
                                         out_hbm.at[pl.ds(bg * CH, CH)],
                                         semw[p])

        def fire_idx(g, _):
            bg = w + g * NW

            @pl.when(bg < NBLK)
            def _():
                pltpu.async_copy(a2b_hbm.at[pl.ds(bg * IB, IB)],
                                 idxa.at[pl.ds(g * IB, IB)], semi)
            return 0

        lax.fori_loop(0, G, fire_idx, 0)

        def drain_fire2(g, _):
            bg = w + g * NW

            @pl.when(bg < NBLK)
            def _():
                pltpu.make_async_copy(a2b_hbm.at[pl.ds(bg * IB, IB)],
                                      idxa.at[pl.ds(g * IB, IB)], semi).wait()
                pltpu.async_copy(b2a_hbm.at[idxa.at[pl.ds(g * IB, IB)]],
                                 idx2a.at[pl.ds(g * IB, IB)], semi2)
            return 0

        lax.fori_loop(0, G, drain_fire2, 0)

        def drain2(g, _):
            bg = w + g * NW

            @pl.when(bg < NBLK)
            def _():
                pltpu.make_async_copy(b2a_hbm.at[idxa.at[pl.ds(g * IB, IB)]],
                                      idx2a.at[pl.ds(g * IB, IB)],
                                      semi2).wait()
            return 0

        lax.fori_loop(0, G, drain2, 0)

        gather_cp(0, 0).start()
        self_cp(w, 0).start()

        def compute(g, p, bg):
            def atom(a, _2):
                base = a * 8
                for t in range(H // 16):
                    sl = pl.ds(t * 16, 16)
                    v = selfv[p][a, sl]
                    for j in range(8):
                        v = v + rows[p][base + j, sl]
                    acc[p][a, sl] = v
                return 0

            lax.fori_loop(0, CH, atom, 0)

            @pl.when(bg == 0)
            def _():
                for t in range(H // 16):
                    acc[p][0, pl.ds(t * 16, 16)] = jnp.zeros((16,),
                                                             jnp.float32)

        def pair(q, _):
            for p in (0, 1):
                g = q * 2 + p
                bg = w + g * NW
                bn = bg + NW

                @pl.when(bn < NBLK)
                def _():
                    gather_cp(g + 1, p ^ 1).start()
                    self_cp(bn, p ^ 1).start()

                @pl.when(bg < NBLK)
                def _():
                    gather_cp(g, p).wait()
                    self_cp(bg, p).wait()

                    @pl.when(g >= 2)
                    def _():
                        wb_cp(bg - 2 * NW, p).wait()

                    compute(g, p, bg)
                    wb_cp(bg, p).start()
            return 0

        lax.fori_loop(0, GP, pair, 0)

        nv = (NBLK - 1 - w) // NW + 1
        for p in (0, 1):
            gl = nv - 1 - (((nv - 1) ^ p) & 1)
            bg = w + gl * NW

            @pl.when(gl >= 0)
            def _():
                wb_cp(bg, p).wait()

    f = pl.kernel(
        body,
        out_type=jax.ShapeDtypeStruct((N, H), jnp.float32),
        mesh=_MESH,
        scratch_types=[
            pltpu.VMEM((G * IB,), jnp.int32),
            pltpu.VMEM((G * IB,), jnp.int32),
            pltpu.VMEM((IB, H), jnp.float32),
            pltpu.VMEM((IB, H), jnp.float32),
            pltpu.VMEM((CH, H), jnp.float32),
            pltpu.VMEM((CH, H), jnp.float32),
            pltpu.VMEM((CH, H), jnp.float32),
            pltpu.VMEM((CH, H), jnp.float32),
            pltpu.SemaphoreType.DMA,
            pltpu.SemaphoreType.DMA,
            pltpu.SemaphoreType.DMA,
            pltpu.SemaphoreType.DMA,
            pltpu.SemaphoreType.DMA,
            pltpu.SemaphoreType.DMA,
            pltpu.SemaphoreType.DMA,
            pltpu.SemaphoreType.DMA,
        ],
    )
    return f(weighted, a2b_flat, b2a)


def _full(shape):
    return pl.BlockSpec(shape, lambda i: (0, 0))


def _dotb(a, b):
    return jnp.dot(a.astype(jnp.bfloat16), b.astype(jnp.bfloat16),
                   preferred_element_type=jnp.float32)


def _pack_tc(y):
    tl = lax.bitcast_convert_type(y[:, :HP], jnp.uint32) + _RND
    th = lax.bitcast_convert_type(y[:, HP:], jnp.uint32) + _RND
    return (tl >> 16) | (th & _HI)


def _unpack_tc(w):
    lo = lax.bitcast_convert_type(w << 16, jnp.float32)
    hi = lax.bitcast_convert_type(w & _HI, jnp.float32)
    return jnp.concatenate([lo, hi], axis=1)


def tc_mm_relu(x, w, m_blk):
    M, K = x.shape
    Kw, Nw = w.shape

    def body(x_ref, w_ref, o_ref):
        o_ref[:] = _pack_tc(jax.nn.relu(_dotb(x_ref[:], w_ref[:])))

    return pl.pallas_call(
        body,
        grid=(M // m_blk,),
        in_specs=[pl.BlockSpec((m_blk, K), lambda i: (i, 0)), _full((Kw, Nw))],
        out_specs=pl.BlockSpec((m_blk, Nw // 2), lambda i: (i, 0)),
        out_shape=jax.ShapeDtypeStruct((M, Nw // 2), jnp.uint32),
    )(x, w)


def tc_mm_residual_relu(msg, h0, w, m_blk):
    M = msg.shape[0]

    def body(m_ref, h0_ref, w_ref, o_ref):
        o_ref[:] = _pack_tc(jax.nn.relu(
            _unpack_tc(h0_ref[:]) + _dotb(_unpack_tc(m_ref[:]), w_ref[:])))

    return pl.pallas_call(
        body,
        grid=(M // m_blk,),
        in_specs=[pl.BlockSpec((m_blk, HP), lambda i: (i, 0)),
                  pl.BlockSpec((m_blk, HP), lambda i: (i, 0)),
                  _full((H, H))],
        out_specs=pl.BlockSpec((m_blk, HP), lambda i: (i, 0)),
        out_shape=jax.ShapeDtypeStruct((M, HP), jnp.uint32),
    )(msg, h0, w)


def tc_atoms(f_atoms, nei, prev, eids, mids, table, W_o, W_vv, W_vc,
             W_conf, b_conf, W_ve_out):
    MB = 1000
    AF = f_atoms.shape[1]
    V = table.shape[0]

    def body(fa, ne, pv, ei, mi, tb, wo, wvv, wvc, wc, bc, wveo,
             atf_o, wgt_o, veo_o):
        pid = pl.program_id(0)
        wo_a = wo[:]
        ne_f = _unpack_tc(ne[:])
        af = jax.nn.relu(_dotb(fa[:], wo_a[:AF]) + _dotb(ne_f, wo_a[AF:]))
        atf = jax.nn.relu(_dotb(pv[:], wvv[:]) + _dotb(af, wvc[:]))

        glob = lax.broadcasted_iota(jnp.int32, (MB, H), 0) + pid * MB
        iota_v = lax.broadcasted_iota(jnp.int32, (MB, V), 1)
        oh_e = (ei[:] == iota_v).astype(jnp.float32)
        ee = _dotb(oh_e, tb[:])
        ee = jnp.where(glob == 0, 0.0, ee)
        oh_m = (mi[:] == iota_v).astype(jnp.float32)
        me = _dotb(oh_m, tb[:])
        me = jnp.where(glob == 0, 0.0, me)

        wc_a = wc[:]
        imp = _dotb(ee, wc_a[:H]) + _dotb(atf, wc_a[H:]) + bc[0, 0]
        atf_o[:] = atf
        wgt_o[:] = imp * ee
        veo_o[:] = _dotb(jax.nn.relu(me), wveo[:])

    return pl.pallas_call(
        body,
        grid=(N // MB,),
        in_specs=[
            pl.BlockSpec((MB, AF), lambda i: (i, 0)),
            pl.BlockSpec((MB, HP), lambda i: (i, 0)),
            pl.BlockSpec((MB, H), lambda i: (i, 0)),
            pl.BlockSpec((MB, 1), lambda i: (i, 0)),
            pl.BlockSpec((MB, 1), lambda i: (i, 0)),
            _full((V, H)),
            _full((AF + H, H)),
            _full((H, H)),
            _full((H, H)),
            _full((2 * H, 1)),
            _full((1, 1)),
            _full((H, H)),
        ],
        out_specs=[pl.BlockSpec((MB, H), lambda i: (i, 0))] * 3,
        out_shape=[jax.ShapeDtypeStruct((N, H), jnp.float32)] * 3,
    )(f_atoms, nei, prev, eids, mids, table, W_o, W_vv, W_vc, W_conf,
      b_conf, W_ve_out)


def tc_head(atf, aee, veo, W_ve, W_ea1, b_ea1, W_ea2, b_ea2,
            W_al1, b_al1, W_al2, b_al2):
    MB = 1000
    MLP = W_ea1.shape[1]
    AOUT = W_al2.shape[1]

    def body(at, ae, vo, wve, wea1, bea1, wea2, bea2, wal1, bal1,
             wal2, bal2, o_ref):
        ve = _dotb(jax.nn.relu(ae[:]), wve[:])
        wea1_a = wea1[:]
        h1 = jax.nn.relu(
            _dotb(at[:], wea1_a[:H]) + _dotb(ve, wea1_a[H:2 * H])
            + _dotb(vo[:], wea1_a[2 * H:]) + bea1[:])
        af2 = jax.nn.relu(_dotb(h1, wea2[:]) + bea2[:])
        s1 = jax.nn.relu(_dotb(af2, wal1[:]) + bal1[:])
        o_ref[:] = _dotb(s1, wal2[:]) + bal2[:]

    return pl.pallas_call(
        body,
        grid=(N // MB,),
        in_specs=[
            pl.BlockSpec((MB, H), lambda i: (i, 0)),
            pl.BlockSpec((MB, H), lambda i: (i, 0)),
            pl.BlockSpec((MB, H), lambda i: (i, 0)),
            _full((H, H)),
            _full((3 * H, MLP)),
            _full((1, MLP)),
            _full((MLP, H)),
            _full((1, H)),
            _full((H, MLP)),
            _full((1, MLP)),
            _full((MLP, AOUT)),
            _full((1, AOUT)),
        ],
        out_specs=pl.BlockSpec((MB, AOUT), lambda i: (i, 0)),
        out_shape=jax.ShapeDtypeStruct((N, AOUT), jnp.float32),
    )(atf, aee, veo, W_ve, W_ea1, b_ea1, W_ea2, b_ea2, W_al1, b_al1,
      W_al2, b_al2)


def kernel(f_atoms, f_bonds, a2b, b2a, b2revb, edit_data, last_edit_mask,
           prev_atom_hiddens, edit_table, W_i, W_h, W_o, W_vv, W_vc, W_conf,
           b_conf, W_ve, W_ve_out, W_ea1, b_ea1, W_ea2, b_ea2, W_al1, b_al1,
           W_al2, b_al2):
    a2b_flat = a2b.reshape(-1).astype(jnp.int32)
    b2a32 = b2a.astype(jnp.int32)
    b2revb32 = b2revb.astype(jnp.int32)

    h0 = tc_mm_relu(f_bonds.astype(jnp.bfloat16), W_i, 2000)
    h = h0
    for _ in range(DEPTH - 1):
        nei = sc_gathersum(h, a2b_flat)
        msg = sc_msg(nei, h, b2a32, b2revb32)
        h = tc_mm_residual_relu(msg, h0, W_h, 2000)
    nei = sc_gathersum(h, a2b_flat)

    atf, weighted, veo = tc_atoms(
        f_atoms, nei, prev_atom_hiddens,
        edit_data.reshape(-1, 1).astype(jnp.int32),
        last_edit_mask.reshape(-1, 1).astype(jnp.int32),
        edit_table, W_o, W_vv, W_vc, W_conf, b_conf.reshape(1, 1), W_ve_out)

    aee = sc_wsum(weighted, a2b_flat, b2a32)

    return tc_head(atf, aee, veo, W_ve, W_ea1, b_ea1.reshape(1, -1),
                   W_ea2, b_ea2.reshape(1, -1), W_al1, b_al1.reshape(1, -1),
                   W_al2, b_al2.reshape(1, -1))


# speedup vs baseline: 1.0904x; 1.0377x over previous
import functools

import jax
import jax.numpy as jnp
import numpy as np
from jax import lax
from jax.experimental import pallas as pl
from jax.experimental.pallas import tpu as pltpu
from jax.experimental.pallas import tpu_sc as plsc

N = 10000
E = 160000
MAXNB = 8
DEPTH = 3
H = 256
NW = 32
HP = H // 2
_HI = np.uint32(0xFFFF0000)
_RND = np.uint32(0x8000)

_MESH = plsc.VectorSubcoreMesh(core_axis_name="c", subcore_axis_name="s",
                               num_cores=2, num_subcores=16)


def _wid():
    return lax.axis_index("s") * 2 + lax.axis_index("c")


def _ceil_div(a, b):
    return (a + b - 1) // b


def sc_gathersum(table, idx_flat):
    CH = 16
    IB = CH * 8
    NBLK = N // CH
    G = _ceil_div(NBLK, NW)
    GP = _ceil_div(G, 2)

    def body(table_hbm, idx_hbm, out_hbm, idxa, rows0, rows1, acc0, acc1,
             semi, semg0, semg1, semw0, semw1):
        w = _wid()
        rows = (rows0, rows1)
        acc = (acc0, acc1)
        semg = (semg0, semg1)
        semw = (semw0, semw1)

        def idx_sl(g):
            return idxa.at[pl.ds(g * IB, IB)]

        def gather_cp(g, p):
            return pltpu.make_async_copy(table_hbm.at[idx_sl(g)],
                                         rows[p], semg[p])

        def wb_cp(bg, p):
            return pltpu.make_async_copy(acc[p],
                                         out_hbm.at[pl.ds(bg * CH, CH)],
                                         semw[p])

        def fire_idx(g, _):
            bg = w + g * NW

            @pl.when(bg < NBLK)
            def _():
                pltpu.async_copy(idx_hbm.at[pl.ds(bg * IB, IB)],
                                 idx_sl(g), semi)
            return 0

        lax.fori_loop(0, G, fire_idx, 0)

        def drain_idx(g, _):
            bg = w + g * NW

            @pl.when(bg < NBLK)
            def _():
                pltpu.make_async_copy(idx_hbm.at[pl.ds(bg * IB, IB)],
                                      idx_sl(g), semi).wait()
            return 0

        lax.fori_loop(0, G, drain_idx, 0)

        gather_cp(0, 0).start()

        def compute(g, p):
            def atom(a, _2):
                base = a * 8
                for t in range(HP // 16):
                    sl = pl.ds(t * 16, 16)
                    s0 = jnp.zeros((16,), jnp.float32)
                    s1 = jnp.zeros((16,), jnp.float32)
                    for j in range(8):
                        wv = rows[p][base + j, sl]
                        s0 = s0 + lax.bitcast_convert_type(wv << 16, jnp.float32)
                        s1 = s1 + lax.bitcast_convert_type(wv & _HI, jnp.float32)
                    u0 = (lax.bitcast_convert_type(s0, jnp.uint32) + _RND) >> 16
                    u1 = (lax.bitcast_convert_type(s1, jnp.uint32) + _RND) & _HI
                    acc[p][a, sl] = u0 | u1
                return 0

            lax.fori_loop(0, CH, atom, 0)

        def pair(q, _):
            for p in (0, 1):
                g = q * 2 + p
                bg = w + g * NW
                bn = bg + NW

                @pl.when(bn < NBLK)
                def _():
                    gather_cp(g + 1, p ^ 1).start()

                @pl.when(bg < NBLK)
                def _():
                    gather_cp(g, p).wait()

                    @pl.when(g >= 2)
                    def _():
                        wb_cp(bg - 2 * NW, p).wait()

                    compute(g, p)
                    wb_cp(bg, p).start()
            return 0

        lax.fori_loop(0, GP, pair, 0)

        nv = (NBLK - 1 - w) // NW + 1
        for p in (0, 1):
            gl = nv - 1 - (((nv - 1) ^ p) & 1)
            bg = w + gl * NW

            @pl.when(gl >= 0)
            def _():
                wb_cp(bg, p).wait()

    f = pl.kernel(
        body,
        out_type=jax.ShapeDtypeStruct((N, HP), jnp.uint32),
        mesh=_MESH,
        scratch_types=[
            pltpu.VMEM((G * IB,), jnp.int32),
            pltpu.VMEM((IB, HP), jnp.uint32),
            pltpu.VMEM((IB, HP), jnp.uint32),
            pltpu.VMEM((CH, HP), jnp.uint32),
            pltpu.VMEM((CH, HP), jnp.uint32),
            pltpu.SemaphoreType.DMA,
            pltpu.SemaphoreType.DMA,
            pltpu.SemaphoreType.DMA,
            pltpu.SemaphoreType.DMA,
            pltpu.SemaphoreType.DMA,
        ],
    )
    return f(table, idx_flat)


def sc_msg(nei, h, b2a, b2revb):
    CB = 64
    NBLK = E // CB
    G = _ceil_div(NBLK, NW)
    GP = _ceil_div(G, 2)

    def body(nei_hbm, h_hbm, b2a_hbm, brev_hbm, out_hbm,
             ia, ib, ra0, ra1, rb0, rb1, oc0, oc1,
             semi, semga0, semga1, semgb0, semgb1, semw0, semw1):
        w = _wid()
        ra = (ra0, ra1)
        rb = (rb0, rb1)
        oc = (oc0, oc1)
        semga = (semga0, semga1)
        semgb = (semgb0, semgb1)
        semw = (semw0, semw1)

        def ga_cp(g, p):
            return pltpu.make_async_copy(
                nei_hbm.at[ia.at[pl.ds(g * CB, CB)]], ra[p], semga[p])

        def gb_cp(g, p):
            return pltpu.make_async_copy(
                h_hbm.at[ib.at[pl.ds(g * CB, CB)]], rb[p], semgb[p])

        def wb_cp(bg, p):
            return pltpu.make_async_copy(oc[p],
                                         out_hbm.at[pl.ds(bg * CB, CB)],
                                         semw[p])

        def fire_idx(g, _):
            bg = w + g * NW

            @pl.when(bg < NBLK)
            def _():
                pltpu.async_copy(b2a_hbm.at[pl.ds(bg * CB, CB)],
                                 ia.at[pl.ds(g * CB, CB)], semi)
                pltpu.async_copy(brev_hbm.at[pl.ds(bg * CB, CB)],
                                 ib.at[pl.ds(g * CB, CB)], semi)
            return 0

        lax.fori_loop(0, G, fire_idx, 0)

        def drain_idx(g, _):
            bg = w + g * NW

            @pl.when(bg < NBLK)
            def _():
                pltpu.make_async_copy(b2a_hbm.at[pl.ds(bg * CB, CB)],
                                      ia.at[pl.ds(g * CB, CB)], semi).wait()
                pltpu.make_async_copy(brev_hbm.at[pl.ds(bg * CB, CB)],
                                      ib.at[pl.ds(g * CB, CB)], semi).wait()
            return 0

        lax.fori_loop(0, G, drain_idx, 0)

        ga_cp(0, 0).start()
        gb_cp(0, 0).start()

        def compute(g, p):
            def bond(a, _2):
                for t in range(HP // 16):
                    sl = pl.ds(t * 16, 16)
                    wa = ra[p][a, sl]
                    wb = rb[p][a, sl]
                    d0 = (lax.bitcast_convert_type(wa << 16, jnp.float32)
                          - lax.bitcast_convert_type(wb << 16, jnp.float32))
                    d1 = (lax.bitcast_convert_type(wa & _HI, jnp.float32)
                          - lax.bitcast_convert_type(wb & _HI, jnp.float32))
                    u0 = (lax.bitcast_convert_type(d0, jnp.uint32) + _RND) >> 16
                    u1 = (lax.bitcast_convert_type(d1, jnp.uint32) + _RND) & _HI
                    oc[p][a, sl] = u0 | u1
                return 0

            lax.fori_loop(0, CB, bond, 0)

        def pair(q, _):
            for p in (0, 1):
                g = q * 2 + p
                bg = w + g * NW
                bn = bg + NW

                @pl.when(bn < NBLK)
                def _():
                    ga_cp(g + 1, p ^ 1).start()
                    gb_cp(g + 1, p ^ 1).start()

                @pl.when(bg < NBLK)
                def _():
                    ga_cp(g, p).wait()
                    gb_cp(g, p).wait()

                    @pl.when(g >= 2)
                    def _():
                        wb_cp(bg - 2 * NW, p).wait()

                    compute(g, p)
                    wb_cp(bg, p).start()
            return 0

        lax.fori_loop(0, GP, pair, 0)

        nv = (NBLK - 1 - w) // NW + 1
        for p in (0, 1):
            gl = nv - 1 - (((nv - 1) ^ p) & 1)
            bg = w + gl * NW

            @pl.when(gl >= 0)
            def _():
                wb_cp(bg, p).wait()

    f = pl.kernel(
        body,
        out_type=jax.ShapeDtypeStruct((E, HP), jnp.uint32),
        mesh=_MESH,
        scratch_types=[
            pltpu.VMEM((G * CB,), jnp.int32),
            pltpu.VMEM((G * CB,), jnp.int32),
            pltpu.VMEM((CB, HP), jnp.uint32),
            pltpu.VMEM((CB, HP), jnp.uint32),
            pltpu.VMEM((CB, HP), jnp.uint32),
            pltpu.VMEM((CB, HP), jnp.uint32),
            pltpu.VMEM((CB, HP), jnp.uint32),
            pltpu.VMEM((CB, HP), jnp.uint32),
            pltpu.SemaphoreType.DMA,
            pltpu.SemaphoreType.DMA,
            pltpu.SemaphoreType.DMA,
            pltpu.SemaphoreType.DMA,
            pltpu.SemaphoreType.DMA,
            pltpu.SemaphoreType.DMA,
            pltpu.SemaphoreType.DMA,
        ],
    )
    return f(nei, h, b2a, b2revb)


def sc_wsum(weighted, a2b_flat, b2a):
    CH = 16
    IB = CH * 8
    NBLK = N // CH
    G = _ceil_div(NBLK, NW)
    GP = _ceil_div(G, 2)

    def body(w_hbm, a2b_hbm, b2a_hbm, out_hbm,
             idxa, idx2a, rows0, rows1, self0, self1, acc0, acc1,
             semi, semi2, semg0, semg1, sems0, sems1, semw0, semw1):
        w = _wid()
        rows = (rows0, rows1)
        selfv = (self0, self1)
        acc = (acc0, acc1)
        semg = (semg0, semg1)
        sems = (sems0, sems1)
        semw = (semw0, semw1)

        def gather_cp(g, p):
            return pltpu.make_async_copy(
                w_hbm.at[idx2a.at[pl.ds(g * IB, IB)]], rows[p], semg[p])

        def self_cp(bg, p):
            return pltpu.make_async_copy(
                w_hbm.at[pl.ds(bg * CH, CH)], selfv[p], sems[p])

        def wb_cp(bg, p):
            return pltpu.make_async_copy(acc[p],
                                         out_hbm.at[pl.ds(bg * CH, CH)],
                                         semw[p])

        def fire_idx(g, _):
            bg = w + g * NW

            @pl.when(bg < NBLK)
            def _():
                pltpu.async_copy(a2b_hbm.at[pl.ds(bg * IB, IB)],
                                 idxa.at[pl.ds(g * IB, IB)], semi)
            return 0

        lax.fori_loop(0, G, fire_idx, 0)

        def drain_fire2(g, _):
            bg = w + g * NW

            @pl.when(bg < NBLK)
            def _():
                pltpu.make_async_copy(a2b_hbm.at[pl.ds(bg * IB, IB)],
                                      idxa.at[pl.ds(g * IB, IB)], semi).wait()
                pltpu.async_copy(b2a_hbm.at[idxa.at[pl.ds(g * IB, IB)]],
                                 idx2a.at[pl.ds(g * IB, IB)], semi2)
            return 0

        lax.fori_loop(0, G, drain_fire2, 0)

        def drain2(g, _):
            bg = w + g * NW

            @pl.when(bg < NBLK)
            def _():
                pltpu.make_async_copy(b2a_hbm.at[idxa.at[pl.ds(g * IB, IB)]],
                                      idx2a.at[pl.ds(g * IB, IB)],
                                      semi2).wait()
            return 0

        lax.fori_loop(0, G, drain2, 0)

        gather_cp(0, 0).start()
        self_cp(w, 0).start()

        def compute(g, p, bg):
            def atom(a, _2):
                base = a * 8
                for t in range(HP // 16):
                    sl = pl.ds(t * 16, 16)
                    wv = selfv[p][a, sl]
                    s0 = lax.bitcast_convert_type(wv << 16, jnp.float32)
                    s1 = lax.bitcast_convert_type(wv & _HI, jnp.float32)
                    for j in range(8):
                        wj = rows[p][base + j, sl]
                        s0 = s0 + lax.bitcast_convert_type(wj << 16,
                                                           jnp.float32)
                        s1 = s1 + lax.bitcast_convert_type(wj & _HI,
                                                           jnp.float32)
                    u0 = (lax.bitcast_convert_type(s0, jnp.uint32)
                          + _RND) >> 16
                    u1 = (lax.bitcast_convert_type(s1, jnp.uint32)
                          + _RND) & _HI
                    acc[p][a, sl] = u0 | u1
                return 0

            lax.fori_loop(0, CH, atom, 0)

            @pl.when(bg == 0)
            def _():
                for t in range(HP // 16):
                    acc[p][0, pl.ds(t * 16, 16)] = jnp.zeros((16,),
                                                             jnp.uint32)

        def pair(q, _):
            for p in (0, 1):
                g = q * 2 + p
                bg = w + g * NW
                bn = bg + NW

                @pl.when(bn < NBLK)
                def _():
                    gather_cp(g + 1, p ^ 1).start()
                    self_cp(bn, p ^ 1).start()

                @pl.when(bg < NBLK)
                def _():
                    gather_cp(g, p).wait()
                    self_cp(bg, p).wait()

                    @pl.when(g >= 2)
                    def _():
                        wb_cp(bg - 2 * NW, p).wait()

                    compute(g, p, bg)
                    wb_cp(bg, p).start()
            return 0

        lax.fori_loop(0, GP, pair, 0)

        nv = (NBLK - 1 - w) // NW + 1
        for p in (0, 1):
            gl = nv - 1 - (((nv - 1) ^ p) & 1)
            bg = w + gl * NW

            @pl.when(gl >= 0)
            def _():
                wb_cp(bg, p).wait()

    f = pl.kernel(
        body,
        out_type=jax.ShapeDtypeStruct((N, HP), jnp.uint32),
        mesh=_MESH,
        scratch_types=[
            pltpu.VMEM((G * IB,), jnp.int32),
            pltpu.VMEM((G * IB,), jnp.int32),
            pltpu.VMEM((IB, HP), jnp.uint32),
            pltpu.VMEM((IB, HP), jnp.uint32),
            pltpu.VMEM((CH, HP), jnp.uint32),
            pltpu.VMEM((CH, HP), jnp.uint32),
            pltpu.VMEM((CH, HP), jnp.uint32),
            pltpu.VMEM((CH, HP), jnp.uint32),
            pltpu.SemaphoreType.DMA,
            pltpu.SemaphoreType.DMA,
            pltpu.SemaphoreType.DMA,
            pltpu.SemaphoreType.DMA,
            pltpu.SemaphoreType.DMA,
            pltpu.SemaphoreType.DMA,
            pltpu.SemaphoreType.DMA,
            pltpu.SemaphoreType.DMA,
        ],
    )
    return f(weighted, a2b_flat, b2a)


def _full(shape):
    return pl.BlockSpec(shape, lambda i: (0, 0))


def _dotb(a, b):
    return jnp.dot(a.astype(jnp.bfloat16), b.astype(jnp.bfloat16),
                   preferred_element_type=jnp.float32)


def _pack_tc(y):
    tl = lax.bitcast_convert_type(y[:, :HP], jnp.uint32) + _RND
    th = lax.bitcast_convert_type(y[:, HP:], jnp.uint32) + _RND
    return (tl >> 16) | (th & _HI)


def _unpack_tc(w):
    lo = lax.bitcast_convert_type(w << 16, jnp.float32)
    hi = lax.bitcast_convert_type(w & _HI, jnp.float32)
    return jnp.concatenate([lo, hi], axis=1)


def tc_mm_relu(x, w, m_blk):
    M, K = x.shape
    Kw, Nw = w.shape

    def body(x_ref, w_ref, o_ref):
        o_ref[:] = _pack_tc(jax.nn.relu(_dotb(x_ref[:], w_ref[:])))

    return pl.pallas_call(
        body,
        grid=(M // m_blk,),
        in_specs=[pl.BlockSpec((m_blk, K), lambda i: (i, 0)), _full((Kw, Nw))],
        out_specs=pl.BlockSpec((m_blk, Nw // 2), lambda i: (i, 0)),
        out_shape=jax.ShapeDtypeStruct((M, Nw // 2), jnp.uint32),
    )(x, w)


def tc_mm_residual_relu(msg, h0, w, m_blk):
    M = msg.shape[0]

    def body(m_ref, h0_ref, w_ref, o_ref):
        o_ref[:] = _pack_tc(jax.nn.relu(
            _unpack_tc(h0_ref[:]) + _dotb(_unpack_tc(m_ref[:]), w_ref[:])))

    return pl.pallas_call(
        body,
        grid=(M // m_blk,),
        in_specs=[pl.BlockSpec((m_blk, HP), lambda i: (i, 0)),
                  pl.BlockSpec((m_blk, HP), lambda i: (i, 0)),
                  _full((H, H))],
        out_specs=pl.BlockSpec((m_blk, HP), lambda i: (i, 0)),
        out_shape=jax.ShapeDtypeStruct((M, HP), jnp.uint32),
    )(msg, h0, w)


def tc_atoms(f_atoms, nei, prev, eids, mids, table, W_o, W_vv, W_vc,
             W_conf, b_conf, W_ve_out):
    MB = 1000
    AF = f_atoms.shape[1]
    V = table.shape[0]

    def body(fa, ne, pv, ei, mi, tb, wo, wvv, wvc, wc, bc, wveo,
             atf_o, wgt_o, veo_o):
        pid = pl.program_id(0)
        wo_a = wo[:]
        ne_f = _unpack_tc(ne[:])
        af = jax.nn.relu(_dotb(fa[:], wo_a[:AF]) + _dotb(ne_f, wo_a[AF:]))
        atf = jax.nn.relu(_dotb(pv[:], wvv[:]) + _dotb(af, wvc[:]))

        glob = lax.broadcasted_iota(jnp.int32, (MB, H), 0) + pid * MB
        iota_v = lax.broadcasted_iota(jnp.int32, (MB, V), 1)
        oh_e = (ei[:] == iota_v).astype(jnp.float32)
        ee = _dotb(oh_e, tb[:])
        ee = jnp.where(glob == 0, 0.0, ee)
        oh_m = (mi[:] == iota_v).astype(jnp.float32)
        me = _dotb(oh_m, tb[:])
        me = jnp.where(glob == 0, 0.0, me)

        wc_a = wc[:]
        imp = _dotb(ee, wc_a[:H]) + _dotb(atf, wc_a[H:]) + bc[0, 0]
        atf_o[:] = atf
        wgt_o[:] = _pack_tc(imp * ee)
        veo_o[:] = _dotb(jax.nn.relu(me), wveo[:])

    return pl.pallas_call(
        body,
        grid=(N // MB,),
        in_specs=[
            pl.BlockSpec((MB, AF), lambda i: (i, 0)),
            pl.BlockSpec((MB, HP), lambda i: (i, 0)),
            pl.BlockSpec((MB, H), lambda i: (i, 0)),
            pl.BlockSpec((MB, 1), lambda i: (i, 0)),
            pl.BlockSpec((MB, 1), lambda i: (i, 0)),
            _full((V, H)),
            _full((AF + H, H)),
            _full((H, H)),
            _full((H, H)),
            _full((2 * H, 1)),
            _full((1, 1)),
            _full((H, H)),
        ],
        out_specs=[pl.BlockSpec((MB, H), lambda i: (i, 0)),
                   pl.BlockSpec((MB, HP), lambda i: (i, 0)),
                   pl.BlockSpec((MB, H), lambda i: (i, 0))],
        out_shape=[jax.ShapeDtypeStruct((N, H), jnp.float32),
                   jax.ShapeDtypeStruct((N, HP), jnp.uint32),
                   jax.ShapeDtypeStruct((N, H), jnp.float32)],
    )(f_atoms, nei, prev, eids, mids, table, W_o, W_vv, W_vc, W_conf,
      b_conf, W_ve_out)


def tc_head(atf, aee, veo, W_ve, W_ea1, b_ea1, W_ea2, b_ea2,
            W_al1, b_al1, W_al2, b_al2):
    MB = 1000
    MLP = W_ea1.shape[1]
    AOUT = W_al2.shape[1]

    def body(at, ae, vo, wve, wea1, bea1, wea2, bea2, wal1, bal1,
             wal2, bal2, o_ref):
        ve = _dotb(jax.nn.relu(_unpack_tc(ae[:])), wve[:])
        wea1_a = wea1[:]
        h1 = jax.nn.relu(
            _dotb(at[:], wea1_a[:H]) + _dotb(ve, wea1_a[H:2 * H])
            + _dotb(vo[:], wea1_a[2 * H:]) + bea1[:])
        af2 = jax.nn.relu(_dotb(h1, wea2[:]) + bea2[:])
        s1 = jax.nn.relu(_dotb(af2, wal1[:]) + bal1[:])
        o_ref[:] = _dotb(s1, wal2[:]) + bal2[:]

    return pl.pallas_call(
        body,
        grid=(N // MB,),
        in_specs=[
            pl.BlockSpec((MB, H), lambda i: (i, 0)),
            pl.BlockSpec((MB, HP), lambda i: (i, 0)),
            pl.BlockSpec((MB, H), lambda i: (i, 0)),
            _full((H, H)),
            _full((3 * H, MLP)),
            _full((1, MLP)),
            _full((MLP, H)),
            _full((1, H)),
            _full((H, MLP)),
            _full((1, MLP)),
            _full((MLP, AOUT)),
            _full((1, AOUT)),
        ],
        out_specs=pl.BlockSpec((MB, AOUT), lambda i: (i, 0)),
        out_shape=jax.ShapeDtypeStruct((N, AOUT), jnp.float32),
    )(atf, aee, veo, W_ve, W_ea1, b_ea1, W_ea2, b_ea2, W_al1, b_al1,
      W_al2, b_al2)


def kernel(f_atoms, f_bonds, a2b, b2a, b2revb, edit_data, last_edit_mask,
           prev_atom_hiddens, edit_table, W_i, W_h, W_o, W_vv, W_vc, W_conf,
           b_conf, W_ve, W_ve_out, W_ea1, b_ea1, W_ea2, b_ea2, W_al1, b_al1,
           W_al2, b_al2):
    a2b_flat = a2b.reshape(-1).astype(jnp.int32)
    b2a32 = b2a.astype(jnp.int32)
    b2revb32 = b2revb.astype(jnp.int32)

    h0 = tc_mm_relu(f_bonds.astype(jnp.bfloat16), W_i, 2000)
    h = h0
    for _ in range(DEPTH - 1):
        nei = sc_gathersum(h, a2b_flat)
        msg = sc_msg(nei, h, b2a32, b2revb32)
        h = tc_mm_residual_relu(msg, h0, W_h, 2000)
    nei = sc_gathersum(h, a2b_flat)

    atf, weighted, veo = tc_atoms(
        f_atoms, nei, prev_atom_hiddens,
        edit_data.reshape(-1, 1).astype(jnp.int32),
        last_edit_mask.reshape(-1, 1).astype(jnp.int32),
        edit_table, W_o, W_vv, W_vc, W_conf, b_conf.reshape(1, 1), W_ve_out)

    aee = sc_wsum(weighted, a2b_flat, b2a32)

    return tc_head(atf, aee, veo, W_ve, W_ea1, b_ea1.reshape(1, -1),
                   W_ea2, b_ea2.reshape(1, -1), W_al1, b_al1.reshape(1, -1),
                   W_al2, b_al2.reshape(1, -1))
